# baseline (speedup 1.0000x reference)
import functools

import jax
import jax.numpy as jnp
from jax import lax
from jax.experimental import pallas as pl
from jax.experimental.pallas import tpu as pltpu
from jax.experimental.pallas import tpu_sc as plsc

B, S, D = 2, 2048, 1024
E, K = 8, 2
T = B * S
BT = 512
BM = 256
PM = ((K * T + E * (BM - 1)) // BM + 1) * BM
NB = PM // BM

_EPS = 1e-5


def _pack_pair(lo_f32, hi_f32):
    lo_b = lax.bitcast_convert_type(
        lo_f32.astype(jnp.bfloat16).astype(jnp.float32), jnp.int32)
    hi_b = lax.bitcast_convert_type(
        hi_f32.astype(jnp.bfloat16).astype(jnp.float32), jnp.int32)
    return jnp.bitwise_or(jnp.bitwise_and(hi_b, jnp.int32(-65536)),
                          lax.shift_right_logical(lo_b, 16))


def _unpack_lo(pk):
    return lax.bitcast_convert_type(jnp.left_shift(pk, 16), jnp.float32)


def _unpack_hi(pk):
    return lax.bitcast_convert_type(
        jnp.bitwise_and(pk, jnp.int32(-65536)), jnp.float32)


def _a_body(x_ref, g1_ref, b1_ref, wg_ref, bg_ref,
            ypk_ref, w1_ref, w2_ref, i1_ref, i2_ref):
    x = x_ref[...]
    m = jnp.mean(x, axis=-1, keepdims=True)
    v = jnp.mean((x - m) ** 2, axis=-1, keepdims=True)
    ln = (x - m) / jnp.sqrt(v + _EPS) * g1_ref[...][None, :] + b1_ref[...][None, :]
    y = x + ln
    ypk_ref[...] = _pack_pair(y[:, :D // 2], y[:, D // 2:])
    logits = jnp.dot(y, wg_ref[...], preferred_element_type=jnp.float32)
    logits = logits + bg_ref[...][None, :]
    eidx = lax.broadcasted_iota(jnp.int32, logits.shape, 1)
    m1 = jnp.max(logits, axis=-1, keepdims=True)
    i1 = jnp.min(jnp.where(logits == m1, eidx, E), axis=-1)
    masked = jnp.where(eidx == i1[:, None], -jnp.inf, logits)
    m2 = jnp.max(masked, axis=-1, keepdims=True)
    i2 = jnp.min(jnp.where(masked == m2, eidx, E), axis=-1)
    sumexp = jnp.sum(jnp.exp(logits - m1), axis=-1)
    w1_ref[...] = 1.0 / sumexp
    w2_ref[...] = jnp.exp(m2[:, 0] - m1[:, 0]) / sumexp
    i1_ref[...] = i1
    i2_ref[...] = i2


def _stage_a(x2d, g1, b1, Wg, bg):
    nblk = T // BT
    return pl.pallas_call(
        _a_body,
        grid=(nblk,),
        in_specs=[
            pl.BlockSpec((BT, D), lambda i: (i, 0)),
            pl.BlockSpec((D,), lambda i: (0,)),
            pl.BlockSpec((D,), lambda i: (0,)),
            pl.BlockSpec((D, E), lambda i: (0, 0)),
            pl.BlockSpec((E,), lambda i: (0,)),
        ],
        out_specs=[
            pl.BlockSpec((BT, D // 2), lambda i: (i, 0)),
            pl.BlockSpec((BT,), lambda i: (i,)),
            pl.BlockSpec((BT,), lambda i: (i,)),
            pl.BlockSpec((BT,), lambda i: (i,)),
            pl.BlockSpec((BT,), lambda i: (i,)),
        ],
        out_shape=[
            jax.ShapeDtypeStruct((T, D // 2), jnp.int32),
            jax.ShapeDtypeStruct((T,), jnp.float32),
            jax.ShapeDtypeStruct((T,), jnp.float32),
            jax.ShapeDtypeStruct((T,), jnp.int32),
            jax.ShapeDtypeStruct((T,), jnp.int32),
        ],
    )(x2d, g1, b1, Wg, bg)


def _excl_prefix_rows(x):
    n = x.shape[0]
    acc = x
    sh = 1
    while sh < n:
        acc = acc + jnp.pad(acc, ((sh, 0), (0, 0)))[:n]
        sh *= 2
    return acc - x


def _routing(i1, i2):
    e_flat = jnp.concatenate([i1, i2])
    oh = (e_flat[:, None] == jnp.arange(E)[None, :]).astype(jnp.int32)
    csum = jnp.cumsum(oh, axis=0)
    rank = jnp.sum((csum - oh) * oh, axis=1)
    counts = csum[-1]
    padded = ((counts + BM - 1) // BM) * BM
    cum = (_excl_prefix_rows(padded[:, None]) + padded[:, None])[:, 0]
    starts = cum - padded
    slot = starts[e_flat] + rank
    blocks = jnp.arange(NB, dtype=jnp.int32) * BM
    be = jnp.clip(jnp.searchsorted(cum, blocks, side='right'), 0, E - 1)
    bvalid = ((blocks < cum[-1]) &
              (blocks - starts[be] < counts[be])).astype(jnp.int32)
    return slot, be.astype(jnp.int32), bvalid


@functools.lru_cache(maxsize=None)
def _make_sc_gather(nrows, ch, nbuf, row_shape, dtype_name):
    dtype = jnp.dtype(dtype_name)
    nc, ns = 2, 16
    nw = nc * ns
    assert nrows % (nw * ch) == 0
    chunks_per_w = nrows // (nw * ch)
    assert chunks_per_w >= nbuf
    mesh = plsc.VectorSubcoreMesh(
        core_axis_name="c", subcore_axis_name="s",
        num_cores=nc, num_subcores=ns)
    buf_shape = (ch,) + row_shape
    scratch = [pltpu.VMEM((chunks_per_w, ch), jnp.int32)]
    scratch += [pltpu.VMEM(buf_shape, dtype) for _ in range(nbuf)]
    scratch += [pltpu.SemaphoreType.DMA for _ in range(2 * nbuf)]

    @functools.partial(
        pl.kernel, mesh=mesh,
        out_type=jax.ShapeDtypeStruct((nrows,) + row_shape, dtype),
        scratch_types=scratch,
    )
    def gather(table_hbm, idx_hbm, out_hbm, idx_v, *bufs_sems):
        bufs = bufs_sems[:nbuf]
        gsem = bufs_sems[nbuf:2 * nbuf]
        wsem = bufs_sems[2 * nbuf:]
        wid = lax.axis_index("s") * nc + lax.axis_index("c")
        pltpu.sync_copy(idx_hbm.at[wid], idx_v)
        g = [None] * nbuf
        w = [None] * nbuf
        for c in range(min(nbuf, chunks_per_w)):
            g[c] = pltpu.async_copy(
                table_hbm.at[idx_v.at[c]], bufs[c], gsem[c])
        for c in range(chunks_per_w):
            b = c % nbuf
            g[b].wait()
            w[b] = pltpu.async_copy(
                bufs[b],
                out_hbm.at[pl.ds((wid * chunks_per_w + c) * ch, ch)],
                wsem[b])
            nxt = c + nbuf
            if nxt < chunks_per_w:
                w[b].wait()
                w[b] = None
                g[b] = pltpu.async_copy(
                    table_hbm.at[idx_v.at[nxt]], bufs[b], gsem[b])
        for b in range(nbuf):
            if w[b] is not None:
                w[b].wait()

    def run(table, idx):
        return gather(table, idx.reshape(nw, chunks_per_w, ch))
    return run


def _gather_o(table, idx):
    return _make_sc_gather(K * T, 64, 2, (D // 2,), "int32")(table, idx)


@functools.lru_cache(maxsize=None)
def _make_sc_scatter():
    dtype = jnp.int32
    dw = D // 2
    nc, ns = 2, 16
    nw = nc * ns
    ch = 32
    tok_per_w = T // nw
    chunks_per_w = tok_per_w // ch
    vregs_per_k = tok_per_w // 16
    mesh = plsc.VectorSubcoreMesh(
        core_axis_name="c", subcore_axis_name="s",
        num_cores=nc, num_subcores=ns)
    scratch = [pltpu.VMEM((chunks_per_w, ch), jnp.int32) for _ in range(2)]
    scratch += [pltpu.VMEM((ch, dw), dtype) for _ in range(2)]
    scratch += [pltpu.SemaphoreType.DMA for _ in range(6)]

    @functools.partial(
        pl.kernel, mesh=mesh,
        out_type=jax.ShapeDtypeStruct((PM, dw), dtype),
        scratch_types=scratch,
    )
    def scatter(table_hbm, idx_hbm, out_hbm, i0_v, i1_v, b0, b1,
                g0, g1, s00, s01, s10, s11):
        wid = lax.axis_index("s") * nc + lax.axis_index("c")
        t0 = wid * tok_per_w
        pltpu.sync_copy(idx_hbm.at[0, wid], i0_v)
        pltpu.sync_copy(idx_hbm.at[1, wid], i1_v)
        bufs, gsem = (b0, b1), (g0, g1)
        wsem = ((s00, s01), (s10, s11))
        g = [None, None]
        w = [[None, None], [None, None]]
        g[0] = pltpu.async_copy(
            table_hbm.at[pl.ds(t0, ch)], bufs[0], gsem[0])
        for c in range(chunks_per_w):
            b = c & 1
            g[b].wait()
            if c + 1 < chunks_per_w:
                nb = 1 - b
                for kk in (0, 1):
                    if w[nb][kk] is not None:
                        w[nb][kk].wait()
                        w[nb][kk] = None
                g[nb] = pltpu.async_copy(
                    table_hbm.at[pl.ds(t0 + (c + 1) * ch, ch)],
                    bufs[nb], gsem[nb])
            w[b][0] = pltpu.async_copy(
                bufs[b], out_hbm.at[i0_v.at[c]], wsem[b][0])
            w[b][1] = pltpu.async_copy(
                bufs[b], out_hbm.at[i1_v.at[c]], wsem[b][1])
        for b in (0, 1):
            for kk in (0, 1):
                if w[b][kk] is not None:
                    w[b][kk].wait()

    def run(table, slot):
        return scatter(table, slot.reshape(2, nw, chunks_per_w, ch))
    return run


def _scatter_x(table, slot):
    return _make_sc_scatter()(table, slot)


def _c_body(be_ref, bv_ref, x_ref, we_ref, b_ref, o_ref, wbf_ref):
    i = pl.program_id(0)

    @pl.when(i == 0)
    def _():
        for e in range(E):
            wbf_ref[e] = we_ref[e].astype(jnp.bfloat16)

    @pl.when(bv_ref[i] == 1)
    def _():
        xi = x_ref[...]
        xb = jnp.concatenate(
            [_unpack_lo(xi).astype(jnp.bfloat16),
             _unpack_hi(xi).astype(jnp.bfloat16)], axis=1)
        e = be_ref[i]
        acc = b_ref[0] + jnp.dot(xb, wbf_ref[e],
                                 preferred_element_type=jnp.float32)
        o_ref[...] = _pack_pair(acc[:, :D // 2], acc[:, D // 2:])


def _stage_c(x_sorted, We, be_bias, block_expert, block_valid):
    grid_spec = pltpu.PrefetchScalarGridSpec(
        num_scalar_prefetch=2,
        grid=(NB,),
        in_specs=[
            pl.BlockSpec((BM, D // 2), lambda i, be, bv: (i, 0)),
            pl.BlockSpec((E, D, D), lambda i, be, bv: (0, 0, 0)),
            pl.BlockSpec((1, 1, D), lambda i, be, bv: (be[i], 0, 0)),
        ],
        out_specs=pl.BlockSpec((BM, D // 2), lambda i, be, bv: (i, 0)),
        scratch_shapes=[pltpu.VMEM((E, D, D), jnp.bfloat16)],
    )
    return pl.pallas_call(
        _c_body,
        grid_spec=grid_spec,
        out_shape=jax.ShapeDtypeStruct((PM, D // 2), jnp.int32),
    )(block_expert, block_valid, x_sorted, We, be_bias.reshape(E, 1, D))


def _d_body(ypk_ref, g0_ref, g1r_ref, w1_ref, w2_ref, g2_ref, b2_ref, o_ref):
    w1 = w1_ref[...][:, None]
    w2 = w2_ref[...][:, None]
    gp0 = g0_ref[...]
    gp1 = g1r_ref[...]
    moe_lo = w1 * _unpack_lo(gp0) + w2 * _unpack_lo(gp1)
    moe_hi = w1 * _unpack_hi(gp0) + w2 * _unpack_hi(gp1)
    s = (jnp.sum(moe_lo, axis=-1, keepdims=True)
         + jnp.sum(moe_hi, axis=-1, keepdims=True))
    m = s / D
    v = (jnp.sum((moe_lo - m) ** 2, axis=-1, keepdims=True)
         + jnp.sum((moe_hi - m) ** 2, axis=-1, keepdims=True)) / D
    r = 1.0 / jnp.sqrt(v + _EPS)
    ypk = ypk_ref[...]
    g2 = g2_ref[...][None, :]
    b2 = b2_ref[...][None, :]
    o_ref[:, :D // 2] = (_unpack_lo(ypk)
                         + (moe_lo - m) * r * g2[:, :D // 2]
                         + b2[:, :D // 2])
    o_ref[:, D // 2:] = (_unpack_hi(ypk)
                         + (moe_hi - m) * r * g2[:, D // 2:]
                         + b2[:, D // 2:])


def _stage_d(y_pk, G, w1, w2, g2, b2):
    nblk = T // BT
    return pl.pallas_call(
        _d_body,
        grid=(nblk,),
        in_specs=[
            pl.BlockSpec((BT, D // 2), lambda i: (i, 0)),
            pl.BlockSpec((BT, D // 2), lambda i: (i, 0)),
            pl.BlockSpec((BT, D // 2), lambda i: (i + T // BT, 0)),
            pl.BlockSpec((BT,), lambda i: (i,)),
            pl.BlockSpec((BT,), lambda i: (i,)),
            pl.BlockSpec((D,), lambda i: (0,)),
            pl.BlockSpec((D,), lambda i: (0,)),
        ],
        out_specs=pl.BlockSpec((BT, D), lambda i: (i, 0)),
        out_shape=jax.ShapeDtypeStruct((T, D), jnp.float32),
    )(y_pk, G, G, w1, w2, g2, b2)


def kernel(x, g1, b1, g2, b2, Wg, bg, We, be):
    x2d = x.reshape(T, D)
    y_pk, w1, w2, i1, i2 = _stage_a(x2d, g1, b1, Wg, bg)
    slot, block_expert, block_valid = _routing(i1, i2)
    x_sorted = _scatter_x(y_pk, slot)
    o_sorted = _stage_c(x_sorted, We, be, block_expert, block_valid)
    G = _gather_o(o_sorted, slot)
    out = _stage_d(y_pk, G, w1, w2, g2, b2)
    return out.reshape(B, S, D)


# speedup vs baseline: 1.0344x; 1.0344x over previous
import functools

import jax
import jax.numpy as jnp
import numpy as np
from jax import lax
from jax.experimental import pallas as pl
from jax.experimental.pallas import tpu as pltpu
from jax.experimental.pallas import tpu_sc as plsc

B, S, D = 2, 2048, 1024
E, K = 8, 2
T = B * S
BT = 512
BM = 256
PM = ((K * T + E * (BM - 1)) // BM + 1) * BM
NB = PM // BM

_EPS = 1e-5


def _pack_pair(lo_f32, hi_f32):
    lo_b = lax.bitcast_convert_type(
        lo_f32.astype(jnp.bfloat16).astype(jnp.float32), jnp.int32)
    hi_b = lax.bitcast_convert_type(
        hi_f32.astype(jnp.bfloat16).astype(jnp.float32), jnp.int32)
    return jnp.bitwise_or(jnp.bitwise_and(hi_b, jnp.int32(-65536)),
                          lax.shift_right_logical(lo_b, 16))


def _unpack_lo(pk):
    return lax.bitcast_convert_type(jnp.left_shift(pk, 16), jnp.float32)


def _unpack_hi(pk):
    return lax.bitcast_convert_type(
        jnp.bitwise_and(pk, jnp.int32(-65536)), jnp.float32)


def _a_body(x_ref, g1_ref, b1_ref, wg_ref, bg_ref,
            ypk_ref, w1_ref, w2_ref, i1_ref, i2_ref):
    x = x_ref[...]
    m = jnp.mean(x, axis=-1, keepdims=True)
    v = jnp.mean((x - m) ** 2, axis=-1, keepdims=True)
    ln = (x - m) / jnp.sqrt(v + _EPS) * g1_ref[...][None, :] + b1_ref[...][None, :]
    y = x + ln
    ypk_ref[...] = _pack_pair(y[:, :D // 2], y[:, D // 2:])
    logits = jnp.dot(y, wg_ref[...], preferred_element_type=jnp.float32)
    logits = logits + bg_ref[...][None, :]
    eidx = lax.broadcasted_iota(jnp.int32, logits.shape, 1)
    m1 = jnp.max(logits, axis=-1, keepdims=True)
    i1 = jnp.min(jnp.where(logits == m1, eidx, E), axis=-1)
    masked = jnp.where(eidx == i1[:, None], -jnp.inf, logits)
    m2 = jnp.max(masked, axis=-1, keepdims=True)
    i2 = jnp.min(jnp.where(masked == m2, eidx, E), axis=-1)
    sumexp = jnp.sum(jnp.exp(logits - m1), axis=-1)
    w1_ref[...] = 1.0 / sumexp
    w2_ref[...] = jnp.exp(m2[:, 0] - m1[:, 0]) / sumexp
    i1_ref[...] = i1
    i2_ref[...] = i2


def _stage_a(x2d, g1, b1, Wg, bg):
    nblk = T // BT
    return pl.pallas_call(
        _a_body,
        grid=(nblk,),
        in_specs=[
            pl.BlockSpec((BT, D), lambda i: (i, 0)),
            pl.BlockSpec((D,), lambda i: (0,)),
            pl.BlockSpec((D,), lambda i: (0,)),
            pl.BlockSpec((D, E), lambda i: (0, 0)),
            pl.BlockSpec((E,), lambda i: (0,)),
        ],
        out_specs=[
            pl.BlockSpec((BT, D // 2), lambda i: (i, 0)),
            pl.BlockSpec((BT,), lambda i: (i,)),
            pl.BlockSpec((BT,), lambda i: (i,)),
            pl.BlockSpec((BT,), lambda i: (i,)),
            pl.BlockSpec((BT,), lambda i: (i,)),
        ],
        out_shape=[
            jax.ShapeDtypeStruct((T, D // 2), jnp.int32),
            jax.ShapeDtypeStruct((T,), jnp.float32),
            jax.ShapeDtypeStruct((T,), jnp.float32),
            jax.ShapeDtypeStruct((T,), jnp.int32),
            jax.ShapeDtypeStruct((T,), jnp.int32),
        ],
    )(x2d, g1, b1, Wg, bg)


def _excl_prefix_rows(x):
    n = x.shape[0]
    acc = x
    sh = 1
    while sh < n:
        acc = acc + jnp.pad(acc, ((sh, 0), (0, 0)))[:n]
        sh *= 2
    return acc - x


def _routing(i1, i2):
    e_flat = jnp.concatenate([i1, i2])
    ohf = (e_flat[:, None] == jnp.arange(E)[None, :]).astype(jnp.float32)
    oh3 = ohf.reshape(K * T // 128, 128, E)
    tril = np.tril(np.ones((128, 128), np.float32), -1)
    local = jnp.einsum('ij,bje->bie', tril, oh3,
                       preferred_element_type=jnp.float32)
    sums = jnp.sum(oh3, axis=1)
    carry = _excl_prefix_rows(sums)
    rank_m = (local + carry[:, None, :]).reshape(K * T, E)
    rank = jnp.sum(rank_m * ohf, axis=1).astype(jnp.int32)
    counts = jnp.sum(sums, axis=0).astype(jnp.int32)
    padded = ((counts + BM - 1) // BM) * BM
    cum = (_excl_prefix_rows(padded[:, None]) + padded[:, None])[:, 0]
    starts = cum - padded
    slot = starts[e_flat] + rank
    blocks = jnp.arange(NB, dtype=jnp.int32) * BM
    be = jnp.clip(jnp.searchsorted(cum, blocks, side='right'), 0, E - 1)
    bvalid = ((blocks < cum[-1]) &
              (blocks - starts[be] < counts[be])).astype(jnp.int32)
    return slot, be.astype(jnp.int32), bvalid


@functools.lru_cache(maxsize=None)
def _make_sc_gather(nrows, ch, nbuf, row_shape, dtype_name):
    dtype = jnp.dtype(dtype_name)
    nc, ns = 2, 16
    nw = nc * ns
    assert nrows % (nw * ch) == 0
    chunks_per_w = nrows // (nw * ch)
    assert chunks_per_w >= nbuf
    mesh = plsc.VectorSubcoreMesh(
        core_axis_name="c", subcore_axis_name="s",
        num_cores=nc, num_subcores=ns)
    buf_shape = (ch,) + row_shape
    scratch = [pltpu.VMEM((chunks_per_w, ch), jnp.int32)]
    scratch += [pltpu.VMEM(buf_shape, dtype) for _ in range(nbuf)]
    scratch += [pltpu.SemaphoreType.DMA for _ in range(2 * nbuf)]

    @functools.partial(
        pl.kernel, mesh=mesh,
        out_type=jax.ShapeDtypeStruct((nrows,) + row_shape, dtype),
        scratch_types=scratch,
    )
    def gather(table_hbm, idx_hbm, out_hbm, idx_v, *bufs_sems):
        bufs = bufs_sems[:nbuf]
        gsem = bufs_sems[nbuf:2 * nbuf]
        wsem = bufs_sems[2 * nbuf:]
        wid = lax.axis_index("s") * nc + lax.axis_index("c")
        pltpu.sync_copy(idx_hbm.at[wid], idx_v)
        g = [None] * nbuf
        w = [None] * nbuf
        for c in range(min(nbuf, chunks_per_w)):
            g[c] = pltpu.async_copy(
                table_hbm.at[idx_v.at[c]], bufs[c], gsem[c])
        for c in range(chunks_per_w):
            b = c % nbuf
            g[b].wait()
            w[b] = pltpu.async_copy(
                bufs[b],
                out_hbm.at[pl.ds((wid * chunks_per_w + c) * ch, ch)],
                wsem[b])
            nxt = c + nbuf
            if nxt < chunks_per_w:
                w[b].wait()
                w[b] = None
                g[b] = pltpu.async_copy(
                    table_hbm.at[idx_v.at[nxt]], bufs[b], gsem[b])
        for b in range(nbuf):
            if w[b] is not None:
                w[b].wait()

    def run(table, idx):
        return gather(table, idx.reshape(nw, chunks_per_w, ch))
    return run


def _gather_o(table, idx):
    return _make_sc_gather(K * T, 64, 2, (D // 2,), "int32")(table, idx)


@functools.lru_cache(maxsize=None)
def _make_sc_scatter():
    dtype = jnp.int32
    dw = D // 2
    nc, ns = 2, 16
    nw = nc * ns
    ch = 32
    tok_per_w = T // nw
    chunks_per_w = tok_per_w // ch
    vregs_per_k = tok_per_w // 16
    mesh = plsc.VectorSubcoreMesh(
        core_axis_name="c", subcore_axis_name="s",
        num_cores=nc, num_subcores=ns)
    scratch = [pltpu.VMEM((chunks_per_w, ch), jnp.int32) for _ in range(2)]
    scratch += [pltpu.VMEM((ch, dw), dtype) for _ in range(2)]
    scratch += [pltpu.SemaphoreType.DMA for _ in range(6)]

    @functools.partial(
        pl.kernel, mesh=mesh,
        out_type=jax.ShapeDtypeStruct((PM, dw), dtype),
        scratch_types=scratch,
    )
    def scatter(table_hbm, idx_hbm, out_hbm, i0_v, i1_v, b0, b1,
                g0, g1, s00, s01, s10, s11):
        wid = lax.axis_index("s") * nc + lax.axis_index("c")
        t0 = wid * tok_per_w
        pltpu.sync_copy(idx_hbm.at[0, wid], i0_v)
        pltpu.sync_copy(idx_hbm.at[1, wid], i1_v)
        bufs, gsem = (b0, b1), (g0, g1)
        wsem = ((s00, s01), (s10, s11))
        g = [None, None]
        w = [[None, None], [None, None]]
        g[0] = pltpu.async_copy(
            table_hbm.at[pl.ds(t0, ch)], bufs[0], gsem[0])
        for c in range(chunks_per_w):
            b = c & 1
            g[b].wait()
            if c + 1 < chunks_per_w:
                nb = 1 - b
                for kk in (0, 1):
                    if w[nb][kk] is not None:
                        w[nb][kk].wait()
                        w[nb][kk] = None
                g[nb] = pltpu.async_copy(
                    table_hbm.at[pl.ds(t0 + (c + 1) * ch, ch)],
                    bufs[nb], gsem[nb])
            w[b][0] = pltpu.async_copy(
                bufs[b], out_hbm.at[i0_v.at[c]], wsem[b][0])
            w[b][1] = pltpu.async_copy(
                bufs[b], out_hbm.at[i1_v.at[c]], wsem[b][1])
        for b in (0, 1):
            for kk in (0, 1):
                if w[b][kk] is not None:
                    w[b][kk].wait()

    def run(table, slot):
        return scatter(table, slot.reshape(2, nw, chunks_per_w, ch))
    return run


def _scatter_x(table, slot):
    return _make_sc_scatter()(table, slot)


def _c_body(be_ref, bv_ref, x_ref, we_ref, b_ref, o_ref, wbf_ref):
    i = pl.program_id(0)

    @pl.when(i == 0)
    def _():
        for e in range(E):
            wbf_ref[e] = we_ref[e].astype(jnp.bfloat16)

    @pl.when(bv_ref[i] == 1)
    def _():
        xi = x_ref[...]
        xb = jnp.concatenate(
            [_unpack_lo(xi).astype(jnp.bfloat16),
             _unpack_hi(xi).astype(jnp.bfloat16)], axis=1)
        e = be_ref[i]
        acc = b_ref[0] + jnp.dot(xb, wbf_ref[e],
                                 preferred_element_type=jnp.float32)
        o_ref[...] = _pack_pair(acc[:, :D // 2], acc[:, D // 2:])


def _stage_c(x_sorted, We, be_bias, block_expert, block_valid):
    grid_spec = pltpu.PrefetchScalarGridSpec(
        num_scalar_prefetch=2,
        grid=(NB,),
        in_specs=[
            pl.BlockSpec((BM, D // 2), lambda i, be, bv: (i, 0)),
            pl.BlockSpec((E, D, D), lambda i, be, bv: (0, 0, 0)),
            pl.BlockSpec((1, 1, D), lambda i, be, bv: (be[i], 0, 0)),
        ],
        out_specs=pl.BlockSpec((BM, D // 2), lambda i, be, bv: (i, 0)),
        scratch_shapes=[pltpu.VMEM((E, D, D), jnp.bfloat16)],
    )
    return pl.pallas_call(
        _c_body,
        grid_spec=grid_spec,
        out_shape=jax.ShapeDtypeStruct((PM, D // 2), jnp.int32),
    )(block_expert, block_valid, x_sorted, We, be_bias.reshape(E, 1, D))


def _d_body(ypk_ref, g0_ref, g1r_ref, w1_ref, w2_ref, g2_ref, b2_ref, o_ref):
    w1 = w1_ref[...][:, None]
    w2 = w2_ref[...][:, None]
    gp0 = g0_ref[...]
    gp1 = g1r_ref[...]
    moe_lo = w1 * _unpack_lo(gp0) + w2 * _unpack_lo(gp1)
    moe_hi = w1 * _unpack_hi(gp0) + w2 * _unpack_hi(gp1)
    s = (jnp.sum(moe_lo, axis=-1, keepdims=True)
         + jnp.sum(moe_hi, axis=-1, keepdims=True))
    m = s / D
    v = (jnp.sum((moe_lo - m) ** 2, axis=-1, keepdims=True)
         + jnp.sum((moe_hi - m) ** 2, axis=-1, keepdims=True)) / D
    r = 1.0 / jnp.sqrt(v + _EPS)
    ypk = ypk_ref[...]
    g2 = g2_ref[...][None, :]
    b2 = b2_ref[...][None, :]
    o_ref[:, :D // 2] = (_unpack_lo(ypk)
                         + (moe_lo - m) * r * g2[:, :D // 2]
                         + b2[:, :D // 2])
    o_ref[:, D // 2:] = (_unpack_hi(ypk)
                         + (moe_hi - m) * r * g2[:, D // 2:]
                         + b2[:, D // 2:])


def _stage_d(y_pk, G, w1, w2, g2, b2):
    nblk = T // BT
    return pl.pallas_call(
        _d_body,
        grid=(nblk,),
        in_specs=[
            pl.BlockSpec((BT, D // 2), lambda i: (i, 0)),
            pl.BlockSpec((BT, D // 2), lambda i: (i, 0)),
            pl.BlockSpec((BT, D // 2), lambda i: (i + T // BT, 0)),
            pl.BlockSpec((BT,), lambda i: (i,)),
            pl.BlockSpec((BT,), lambda i: (i,)),
            pl.BlockSpec((D,), lambda i: (0,)),
            pl.BlockSpec((D,), lambda i: (0,)),
        ],
        out_specs=pl.BlockSpec((BT, D), lambda i: (i, 0)),
        out_shape=jax.ShapeDtypeStruct((T, D), jnp.float32),
    )(y_pk, G, G, w1, w2, g2, b2)


def kernel(x, g1, b1, g2, b2, Wg, bg, We, be):
    x2d = x.reshape(T, D)
    y_pk, w1, w2, i1, i2 = _stage_a(x2d, g1, b1, Wg, bg)
    slot, block_expert, block_valid = _routing(i1, i2)
    x_sorted = _scatter_x(y_pk, slot)
    o_sorted = _stage_c(x_sorted, We, be, block_expert, block_valid)
    G = _gather_o(o_sorted, slot)
    out = _stage_d(y_pk, G, w1, w2, g2, b2)
    return out.reshape(B, S, D)
